# baseline (speedup 1.0000x reference)
import functools

import jax
import jax.numpy as jnp
from jax import lax
from jax.experimental import pallas as pl
from jax.experimental.pallas import tpu as pltpu
from jax.experimental.pallas import tpu_sc as plsc

N = 10000
E = 160000
SEQ_D = 9216
H = 32
EPS = 1e-5

P = 4
NP = N // P
BLKP = 128
CHUNK = 1000
NW = 32
NCHUNK = E // CHUNK
ROWS_PER_S = 640


def _gelu(x):
    return 0.5 * x * (1.0 + lax.erf(x * (2.0 ** -0.5)))


def _fold_stats(a):
    s1 = jnp.sum(a, axis=0, keepdims=True)
    s2 = jnp.sum(a * a, axis=0, keepdims=True)
    s1 = s1[:, :32] + s1[:, 32:64] + s1[:, 64:96] + s1[:, 96:]
    s2 = s2[:, :32] + s2[:, 32:64] + s2[:, 64:96] + s2[:, 96:]
    s1 = jnp.concatenate([s1, s1, s1, s1], axis=1)
    s2 = jnp.concatenate([s2, s2, s2, s2], axis=1)
    m = s1 / N
    v = s2 / N - m * m
    return m, v


def _bn_gelu(a, g4, b4):
    m, v = _fold_stats(a)
    return _gelu((a - m) / jnp.sqrt(v + EPS) * g4 + b4)


def _tile4(w):
    w = jnp.reshape(w, (1, H))
    return jnp.concatenate([w, w, w, w], axis=1)


def _gemm_body(seq_ref, we_ref, a_ref):
    a_ref[...] = jnp.dot(seq_ref[...].astype(jnp.bfloat16), we_ref[...],
                         preferred_element_type=jnp.float32)


def _encoder_gemm(seq_r, We_bd):
    return pl.pallas_call(
        _gemm_body,
        grid=(pl.cdiv(NP, BLKP),),
        in_specs=[
            pl.BlockSpec((BLKP, P * SEQ_D), lambda i: (i, 0)),
            pl.BlockSpec((P * SEQ_D, 128), lambda i: (0, 0)),
        ],
        out_specs=pl.BlockSpec((BLKP, 128), lambda i: (i, 0)),
        out_shape=jax.ShapeDtypeStruct((NP, 128), jnp.float32),
    )(seq_r, We_bd)


def _mid_body(a_ref, x_ref, p_ref, wx_ref, gx_ref, bex_ref, wp_ref, gp_ref,
              bep_ref, ge_ref, bee_ref, wft_ref, wfb_ref, gf_ref, bef_ref,
              h2_ref):
    h_e = _bn_gelu(a_ref[...], _tile4(ge_ref[...]), _tile4(bee_ref[...]))

    xv = x_ref[...]
    mx = jnp.sum(xv) / (N * H)
    vx = jnp.sum((xv - mx) ** 2) / (N * H)
    wx = _tile4(wx_ref[...])
    h_x = _gelu((xv - mx) * wx / jnp.sqrt(vx * wx * wx + EPS)
                * _tile4(gx_ref[...]) + _tile4(bex_ref[...]))

    pv = p_ref[...]
    mp = jnp.sum(pv) / (N * H)
    vp = jnp.sum((pv - mp) ** 2) / (N * H)
    wp = _tile4(wp_ref[...])
    h_p = _gelu((pv - mp) * wp / jnp.sqrt(vp * wp * wp + EPS)
                * _tile4(gp_ref[...]) + _tile4(bep_ref[...]))

    h = h_x + h_e
    B = (jnp.dot(h, wft_ref[...], preferred_element_type=jnp.float32)
         + jnp.dot(h_p, wfb_ref[...], preferred_element_type=jnp.float32))
    h2_ref[...] = _bn_gelu(B, _tile4(gf_ref[...]), _tile4(bef_ref[...]))


def _mid_stage(A, xp, pp, W_x, g_x, be_x, W_p, g_p, be_p, g_e, be_e,
               Wf_t, Wf_b, g_f, be_f):
    return pl.pallas_call(
        _mid_body,
        out_shape=jax.ShapeDtypeStruct((NP, 128), jnp.float32),
    )(A, xp, pp, W_x, g_x, be_x, W_p, g_p, be_p, g_e, be_e,
      Wf_t, Wf_b, g_f, be_f)


def _sc_scatter(h2, edge_index, zeros):
    mesh = plsc.VectorSubcoreMesh(core_axis_name="c", subcore_axis_name="s")

    @functools.partial(
        pl.kernel,
        out_type=(jax.ShapeDtypeStruct((N, H), jnp.float32),
                  jax.ShapeDtypeStruct((N, H), jnp.float32)),
        mesh=mesh,
        scratch_types=[
            pltpu.VMEM((CHUNK,), jnp.int32),
            pltpu.VMEM((CHUNK,), jnp.int32),
            pltpu.VMEM((CHUNK, H), jnp.float32),
            pltpu.VMEM_SHARED((N, H), jnp.float32),
            pltpu.SemaphoreType.DMA,
        ],
        compiler_params=pltpu.CompilerParams(use_tc_tiling_on_sc=False),
    )
    def k(h2_hbm, ei_hbm, zeros_hbm, out0_hbm, out1_hbm,
          idx_v, dst_v, rows_v, agg_sh, sem):
        c = lax.axis_index("c")
        s = lax.axis_index("s")
        w = c * 16 + s

        def _row_copy(src_ref, dst_ref):
            sr = ROWS_PER_S

            @pl.when(s < 15)
            def _():
                pltpu.sync_copy(src_ref.at[pl.ds(s * sr, sr)],
                                dst_ref.at[pl.ds(s * sr, sr)])

            @pl.when(s == 15)
            def _():
                pltpu.sync_copy(src_ref.at[pl.ds(15 * sr, N - 15 * sr)],
                                dst_ref.at[pl.ds(15 * sr, N - 15 * sr)])

        _row_copy(zeros_hbm, agg_sh)
        plsc.subcore_barrier()

        def step(t):
            base = pl.multiple_of(t * CHUNK, 8)
            pltpu.sync_copy(ei_hbm.at[0, pl.ds(base, CHUNK)], idx_v)
            pltpu.async_copy(h2_hbm.at[idx_v], rows_v, sem).wait()
            pltpu.sync_copy(ei_hbm.at[1, pl.ds(base, CHUNK)], dst_v)
            pltpu.sync_copy(rows_v, agg_sh.at[dst_v], add=True)

        def body(j, carry):
            step(j * NW + w)
            return carry
        lax.fori_loop(0, NCHUNK // NW, body, 0)

        plsc.subcore_barrier()

        @pl.when(c == 0)
        def _():
            _row_copy(agg_sh, out0_hbm)

        @pl.when(c == 1)
        def _():
            _row_copy(agg_sh, out1_hbm)

    return k(h2, edge_index, zeros)


def _out_body(agg0_ref, agg1_ref, h2_ref, wl_ref, wr_ref, gc_ref, bec_ref,
              wo_ref, bo_ref, z_ref, out_ref):
    agg = agg0_ref[...] + agg1_ref[...]
    h2 = h2_ref[...]
    z0 = (jnp.dot(agg, wl_ref[...], preferred_element_type=jnp.float32)
          + jnp.dot(h2, wr_ref[...], preferred_element_type=jnp.float32))
    z = _bn_gelu(z0, _tile4(gc_ref[...]), _tile4(bec_ref[...]))
    z_ref[...] = z
    out_ref[...] = jnp.maximum(
        jnp.dot(z, wo_ref[...], preferred_element_type=jnp.float32)
        + bo_ref[...].reshape(1, 1), 0.0)


def _out_stage(agg0, agg1, h2, Wl_bd, Wr_bd, g_c, be_c, Wo_bd, b_o):
    return pl.pallas_call(
        _out_body,
        out_shape=(jax.ShapeDtypeStruct((NP, 128), jnp.float32),
                   jax.ShapeDtypeStruct((NP, P), jnp.float32)),
    )(agg0, agg1, h2, Wl_bd, Wr_bd, g_c, be_c, Wo_bd, b_o)


def kernel(x, seq, pause, edge_index, W_x, b_x, g_x, be_x, W_p, b_p, g_p, be_p,
           W_e, b_e, g_e, be_e, W_f, b_f, g_f, be_f, W_l, b_l, W_r, g_c, be_c,
           W_o, b_o):
    eye4 = jnp.eye(P, dtype=jnp.float32)
    We_bd = jnp.kron(eye4.astype(jnp.bfloat16),
                     W_e.astype(jnp.bfloat16))
    Wf_t_bd = jnp.kron(eye4, W_f[:H])
    Wf_b_bd = jnp.kron(eye4, W_f[H:])
    Wl_bd = jnp.kron(eye4, W_l)
    Wr_bd = jnp.kron(eye4, W_r)
    Wo_bd = jnp.kron(eye4, W_o)

    seq_r = seq.reshape(NP, P * SEQ_D)
    A = _encoder_gemm(seq_r, We_bd)

    xp = jnp.broadcast_to(x.reshape(NP, P, 1), (NP, P, H)).reshape(NP, 128)
    pp = jnp.broadcast_to(pause.reshape(NP, P, 1), (NP, P, H)).reshape(NP, 128)

    h2 = _mid_stage(A, xp, pp, W_x, g_x, be_x, W_p, g_p, be_p, g_e, be_e,
                    Wf_t_bd, Wf_b_bd, g_f, be_f)

    zeros = jnp.zeros((N, H), jnp.float32)
    agg0, agg1 = _sc_scatter(h2.reshape(N, H), edge_index, zeros)

    z, out = _out_stage(agg0.reshape(NP, 128), agg1.reshape(NP, 128), h2,
                        Wl_bd, Wr_bd, g_c, be_c, Wo_bd, b_o)
    return (out.reshape(N, 1), z.reshape(N, H))


# speedup vs baseline: 2.9864x; 2.9864x over previous
import functools

import jax
import jax.numpy as jnp
from jax import lax
from jax.experimental import pallas as pl
from jax.experimental.pallas import tpu as pltpu
from jax.experimental.pallas import tpu_sc as plsc

N = 10000
E = 160000
SEQ_D = 9216
H = 32
EPS = 1e-5

P = 4
NP = N // P
BLKP = 128
CHUNK = 1000
NW = 32
NCHUNK = E // CHUNK
ROWS_PER_S = 640


def _gelu(x):
    return 0.5 * x * (1.0 + lax.erf(x * (2.0 ** -0.5)))


def _fold_stats(a):
    s1 = jnp.sum(a, axis=0, keepdims=True)
    s2 = jnp.sum(a * a, axis=0, keepdims=True)
    s1 = s1[:, :32] + s1[:, 32:64] + s1[:, 64:96] + s1[:, 96:]
    s2 = s2[:, :32] + s2[:, 32:64] + s2[:, 64:96] + s2[:, 96:]
    s1 = jnp.concatenate([s1, s1, s1, s1], axis=1)
    s2 = jnp.concatenate([s2, s2, s2, s2], axis=1)
    m = s1 / N
    v = s2 / N - m * m
    return m, v


def _bn_gelu(a, g4, b4):
    m, v = _fold_stats(a)
    return _gelu((a - m) / jnp.sqrt(v + EPS) * g4 + b4)


def _tile4(w):
    w = jnp.reshape(w, (1, H))
    return jnp.concatenate([w, w, w, w], axis=1)


def _gemm_body(seq_ref, we_ref, a_ref):
    a_ref[...] = jnp.dot(seq_ref[...].astype(jnp.bfloat16), we_ref[...],
                         preferred_element_type=jnp.float32)


BLK = 400


def _encoder_gemm(seq, We_b16):
    return pl.pallas_call(
        _gemm_body,
        grid=(N // BLK,),
        in_specs=[
            pl.BlockSpec((BLK, SEQ_D), lambda i: (i, 0)),
            pl.BlockSpec((SEQ_D, H), lambda i: (0, 0)),
        ],
        out_specs=pl.BlockSpec((BLK, H), lambda i: (i, 0)),
        out_shape=jax.ShapeDtypeStruct((N, H), jnp.float32),
    )(seq, We_b16)


def _mid_body(a_ref, x_ref, p_ref, wx_ref, gx_ref, bex_ref, wp_ref, gp_ref,
              bep_ref, ge_ref, bee_ref, wft_ref, wfb_ref, gf_ref, bef_ref,
              h2_ref):
    h_e = _bn_gelu(a_ref[...], _tile4(ge_ref[...]), _tile4(bee_ref[...]))

    xv = x_ref[...]
    mx = jnp.sum(xv) / (N * H)
    vx = jnp.sum((xv - mx) ** 2) / (N * H)
    wx = _tile4(wx_ref[...])
    h_x = _gelu((xv - mx) * wx / jnp.sqrt(vx * wx * wx + EPS)
                * _tile4(gx_ref[...]) + _tile4(bex_ref[...]))

    pv = p_ref[...]
    mp = jnp.sum(pv) / (N * H)
    vp = jnp.sum((pv - mp) ** 2) / (N * H)
    wp = _tile4(wp_ref[...])
    h_p = _gelu((pv - mp) * wp / jnp.sqrt(vp * wp * wp + EPS)
                * _tile4(gp_ref[...]) + _tile4(bep_ref[...]))

    h = h_x + h_e
    B = (jnp.dot(h, wft_ref[...], preferred_element_type=jnp.float32)
         + jnp.dot(h_p, wfb_ref[...], preferred_element_type=jnp.float32))
    h2_ref[...] = _bn_gelu(B, _tile4(gf_ref[...]), _tile4(bef_ref[...]))


def _mid_stage(A, xp, pp, W_x, g_x, be_x, W_p, g_p, be_p, g_e, be_e,
               Wf_t, Wf_b, g_f, be_f):
    return pl.pallas_call(
        _mid_body,
        out_shape=jax.ShapeDtypeStruct((NP, 128), jnp.float32),
    )(A, xp, pp, W_x, g_x, be_x, W_p, g_p, be_p, g_e, be_e,
      Wf_t, Wf_b, g_f, be_f)


def _sc_scatter(h2, edge_index, zeros):
    mesh = plsc.VectorSubcoreMesh(core_axis_name="c", subcore_axis_name="s")

    @functools.partial(
        pl.kernel,
        out_type=(jax.ShapeDtypeStruct((N, H), jnp.float32),
                  jax.ShapeDtypeStruct((N, H), jnp.float32)),
        mesh=mesh,
        scratch_types=[
            pltpu.VMEM((CHUNK,), jnp.int32),
            pltpu.VMEM((CHUNK,), jnp.int32),
            pltpu.VMEM((CHUNK, H), jnp.float32),
            pltpu.VMEM_SHARED((N, H), jnp.float32),
            pltpu.SemaphoreType.DMA,
        ],
        compiler_params=pltpu.CompilerParams(use_tc_tiling_on_sc=False),
    )
    def k(h2_hbm, ei_hbm, zeros_hbm, out0_hbm, out1_hbm,
          idx_v, dst_v, rows_v, agg_sh, sem):
        c = lax.axis_index("c")
        s = lax.axis_index("s")
        w = c * 16 + s

        def _row_copy(src_ref, dst_ref):
            sr = ROWS_PER_S

            @pl.when(s < 15)
            def _():
                pltpu.sync_copy(src_ref.at[pl.ds(s * sr, sr)],
                                dst_ref.at[pl.ds(s * sr, sr)])

            @pl.when(s == 15)
            def _():
                pltpu.sync_copy(src_ref.at[pl.ds(15 * sr, N - 15 * sr)],
                                dst_ref.at[pl.ds(15 * sr, N - 15 * sr)])

        _row_copy(zeros_hbm, agg_sh)
        plsc.subcore_barrier()

        def step(t):
            base = pl.multiple_of(t * CHUNK, 8)
            pltpu.sync_copy(ei_hbm.at[0, pl.ds(base, CHUNK)], idx_v)
            pltpu.async_copy(h2_hbm.at[idx_v], rows_v, sem).wait()
            pltpu.sync_copy(ei_hbm.at[1, pl.ds(base, CHUNK)], dst_v)
            pltpu.sync_copy(rows_v, agg_sh.at[dst_v], add=True)

        def body(j, carry):
            step(j * NW + w)
            return carry
        lax.fori_loop(0, NCHUNK // NW, body, 0)

        plsc.subcore_barrier()

        @pl.when(c == 0)
        def _():
            _row_copy(agg_sh, out0_hbm)

        @pl.when(c == 1)
        def _():
            _row_copy(agg_sh, out1_hbm)

    return k(h2, edge_index, zeros)


def _out_body(agg0_ref, agg1_ref, h2_ref, wl_ref, wr_ref, gc_ref, bec_ref,
              wo_ref, bo_ref, z_ref, out_ref):
    agg = agg0_ref[...] + agg1_ref[...]
    h2 = h2_ref[...]
    z0 = (jnp.dot(agg, wl_ref[...], preferred_element_type=jnp.float32)
          + jnp.dot(h2, wr_ref[...], preferred_element_type=jnp.float32))
    z = _bn_gelu(z0, _tile4(gc_ref[...]), _tile4(bec_ref[...]))
    z_ref[...] = z
    out_ref[...] = jnp.maximum(
        jnp.dot(z, wo_ref[...], preferred_element_type=jnp.float32)
        + bo_ref[...].reshape(1, 1), 0.0)


def _out_stage(agg0, agg1, h2, Wl_bd, Wr_bd, g_c, be_c, Wo_bd, b_o):
    return pl.pallas_call(
        _out_body,
        out_shape=(jax.ShapeDtypeStruct((NP, 128), jnp.float32),
                   jax.ShapeDtypeStruct((NP, P), jnp.float32)),
    )(agg0, agg1, h2, Wl_bd, Wr_bd, g_c, be_c, Wo_bd, b_o)


def kernel(x, seq, pause, edge_index, W_x, b_x, g_x, be_x, W_p, b_p, g_p, be_p,
           W_e, b_e, g_e, be_e, W_f, b_f, g_f, be_f, W_l, b_l, W_r, g_c, be_c,
           W_o, b_o):
    eye4 = jnp.eye(P, dtype=jnp.float32)
    Wf_t_bd = jnp.kron(eye4, W_f[:H])
    Wf_b_bd = jnp.kron(eye4, W_f[H:])
    Wl_bd = jnp.kron(eye4, W_l)
    Wr_bd = jnp.kron(eye4, W_r)
    Wo_bd = jnp.kron(eye4, W_o)

    A = _encoder_gemm(seq, W_e.astype(jnp.bfloat16)).reshape(NP, 128)

    xp = jnp.broadcast_to(x.reshape(NP, P, 1), (NP, P, H)).reshape(NP, 128)
    pp = jnp.broadcast_to(pause.reshape(NP, P, 1), (NP, P, H)).reshape(NP, 128)

    h2 = _mid_stage(A, xp, pp, W_x, g_x, be_x, W_p, g_p, be_p, g_e, be_e,
                    Wf_t_bd, Wf_b_bd, g_f, be_f)

    zeros = jnp.zeros((N, H), jnp.float32)
    agg0, agg1 = _sc_scatter(h2.reshape(N, H), edge_index, zeros)

    z, out = _out_stage(agg0.reshape(NP, 128), agg1.reshape(NP, 128), h2,
                        Wl_bd, Wr_bd, g_c, be_c, Wo_bd, b_o)
    return (out.reshape(N, 1), z.reshape(N, H))


# speedup vs baseline: 3.1192x; 1.0445x over previous
import functools

import jax
import jax.numpy as jnp
from jax import lax
from jax.experimental import pallas as pl
from jax.experimental.pallas import tpu as pltpu
from jax.experimental.pallas import tpu_sc as plsc

N = 10000
E = 160000
SEQ_D = 9216
H = 32
EPS = 1e-5

P = 4
NP = N // P
CHUNK = 1000
NW = 32
NCHUNK = E // CHUNK
ROWS_PER_S = 640


def _gelu(x):
    return 0.5 * x * (1.0 + lax.erf(x * (2.0 ** -0.5)))


def _fold_stats(a):
    s1 = jnp.sum(a, axis=0, keepdims=True)
    s2 = jnp.sum(a * a, axis=0, keepdims=True)
    s1 = s1[:, :32] + s1[:, 32:64] + s1[:, 64:96] + s1[:, 96:]
    s2 = s2[:, :32] + s2[:, 32:64] + s2[:, 64:96] + s2[:, 96:]
    s1 = jnp.concatenate([s1, s1, s1, s1], axis=1)
    s2 = jnp.concatenate([s2, s2, s2, s2], axis=1)
    m = s1 / N
    v = s2 / N - m * m
    return m, v


def _bn_gelu(a, g4, b4):
    m, v = _fold_stats(a)
    return _gelu((a - m) / jnp.sqrt(v + EPS) * g4 + b4)


def _tile4(w):
    w = jnp.reshape(w, (1, H))
    return jnp.concatenate([w, w, w, w], axis=1)


def _gemm_body(seq_ref, we_ref, a_ref):
    a_ref[...] = jnp.dot(seq_ref[...].astype(jnp.bfloat16), we_ref[...],
                         preferred_element_type=jnp.float32)


BLK = 400


def _encoder_gemm(seq, We_b16):
    return pl.pallas_call(
        _gemm_body,
        grid=(N // BLK,),
        in_specs=[
            pl.BlockSpec((BLK, SEQ_D), lambda i: (i, 0)),
            pl.BlockSpec((SEQ_D, H), lambda i: (0, 0)),
        ],
        out_specs=pl.BlockSpec((BLK, H), lambda i: (i, 0)),
        out_shape=jax.ShapeDtypeStruct((N, H), jnp.float32),
    )(seq, We_b16)


def _mid_body(a_ref, x_ref, p_ref, wx_ref, gx_ref, bex_ref, wp_ref, gp_ref,
              bep_ref, ge_ref, bee_ref, wft_ref, wfb_ref, gf_ref, bef_ref,
              h2_ref):
    h_e = _bn_gelu(a_ref[...], _tile4(ge_ref[...]), _tile4(bee_ref[...]))

    xv = x_ref[...]
    mx = jnp.sum(xv) / (N * H)
    vx = jnp.sum((xv - mx) ** 2) / (N * H)
    wx = _tile4(wx_ref[...])
    h_x = _gelu((xv - mx) * wx / jnp.sqrt(vx * wx * wx + EPS)
                * _tile4(gx_ref[...]) + _tile4(bex_ref[...]))

    pv = p_ref[...]
    mp = jnp.sum(pv) / (N * H)
    vp = jnp.sum((pv - mp) ** 2) / (N * H)
    wp = _tile4(wp_ref[...])
    h_p = _gelu((pv - mp) * wp / jnp.sqrt(vp * wp * wp + EPS)
                * _tile4(gp_ref[...]) + _tile4(bep_ref[...]))

    h = h_x + h_e
    B = (jnp.dot(h, wft_ref[...], preferred_element_type=jnp.float32)
         + jnp.dot(h_p, wfb_ref[...], preferred_element_type=jnp.float32))
    h2_ref[...] = _bn_gelu(B, _tile4(gf_ref[...]), _tile4(bef_ref[...]))


def _mid_stage(A, xp, pp, W_x, g_x, be_x, W_p, g_p, be_p, g_e, be_e,
               Wf_t, Wf_b, g_f, be_f):
    return pl.pallas_call(
        _mid_body,
        out_shape=jax.ShapeDtypeStruct((NP, 128), jnp.float32),
    )(A, xp, pp, W_x, g_x, be_x, W_p, g_p, be_p, g_e, be_e,
      Wf_t, Wf_b, g_f, be_f)


def _sc_scatter(h2, edge_index, zeros):
    mesh = plsc.VectorSubcoreMesh(core_axis_name="c", subcore_axis_name="s")

    @functools.partial(
        pl.kernel,
        out_type=(jax.ShapeDtypeStruct((N, H), jnp.float32),
                  jax.ShapeDtypeStruct((N, H), jnp.float32)),
        mesh=mesh,
        scratch_types=[
            pltpu.VMEM((NCHUNK // NW, CHUNK), jnp.int32),
            pltpu.VMEM((NCHUNK // NW, CHUNK), jnp.int32),
            pltpu.VMEM((CHUNK, H), jnp.float32),
            pltpu.VMEM((CHUNK, H), jnp.float32),
            pltpu.VMEM_SHARED((N, H), jnp.float32),
            pltpu.SemaphoreType.DMA,
            pltpu.SemaphoreType.DMA,
            pltpu.SemaphoreType.DMA,
            pltpu.SemaphoreType.DMA,
            pltpu.SemaphoreType.DMA,
        ],
        compiler_params=pltpu.CompilerParams(use_tc_tiling_on_sc=False),
    )
    def k(h2_hbm, ei_hbm, zeros_hbm, out0_hbm, out1_hbm,
          idx_s2, idx_d2, rows0, rows1, agg_sh,
          semi, semg0, semg1, sems0, sems1):
        c = lax.axis_index("c")
        s = lax.axis_index("s")
        w = c * 16 + s

        def _row_copy(src_ref, dst_ref):
            sr = ROWS_PER_S

            @pl.when(s < 15)
            def _():
                pltpu.sync_copy(src_ref.at[pl.ds(s * sr, sr)],
                                dst_ref.at[pl.ds(s * sr, sr)])

            @pl.when(s == 15)
            def _():
                pltpu.sync_copy(src_ref.at[pl.ds(15 * sr, N - 15 * sr)],
                                dst_ref.at[pl.ds(15 * sr, N - 15 * sr)])

        nj = NCHUNK // NW
        hidx = []
        for j in range(nj):
            base = pl.multiple_of((j * NW + w) * CHUNK, 8)
            hidx.append(pltpu.async_copy(
                ei_hbm.at[0, pl.ds(base, CHUNK)], idx_s2.at[j], semi))
            hidx.append(pltpu.async_copy(
                ei_hbm.at[1, pl.ds(base, CHUNK)], idx_d2.at[j], semi))

        _row_copy(zeros_hbm, agg_sh)
        for h in hidx:
            h.wait()
        plsc.subcore_barrier()

        NB = 2
        rows = (rows0, rows1)
        semg = (semg0, semg1)
        sems = (sems0, sems1)
        hg = [None] * nj
        hs = [None] * nj
        for j in range(nj):
            b = j % NB
            if j >= NB:
                hs[j - NB].wait()
            hg[j] = pltpu.async_copy(h2_hbm.at[idx_s2.at[j]], rows[b], semg[b])
            if j >= 1:
                hg[j - 1].wait()
                hs[j - 1] = pltpu.async_copy(
                    rows[(j - 1) % NB], agg_sh.at[idx_d2.at[j - 1]],
                    sems[(j - 1) % NB], add=True)
        hg[nj - 1].wait()
        hs[nj - 1] = pltpu.async_copy(
            rows[(nj - 1) % NB], agg_sh.at[idx_d2.at[nj - 1]],
            sems[(nj - 1) % NB], add=True)
        for j in range(max(0, nj - NB + 1), nj):
            hs[j].wait()

        plsc.subcore_barrier()

        @pl.when(c == 0)
        def _():
            _row_copy(agg_sh, out0_hbm)

        @pl.when(c == 1)
        def _():
            _row_copy(agg_sh, out1_hbm)

    return k(h2, edge_index, zeros)


def _out_body(agg0_ref, agg1_ref, h2_ref, wl_ref, wr_ref, gc_ref, bec_ref,
              wo_ref, bo_ref, z_ref, out_ref):
    agg = agg0_ref[...] + agg1_ref[...]
    h2 = h2_ref[...]
    z0 = (jnp.dot(agg, wl_ref[...], preferred_element_type=jnp.float32)
          + jnp.dot(h2, wr_ref[...], preferred_element_type=jnp.float32))
    z = _bn_gelu(z0, _tile4(gc_ref[...]), _tile4(bec_ref[...]))
    z_ref[...] = z
    out_ref[...] = jnp.maximum(
        jnp.dot(z, wo_ref[...], preferred_element_type=jnp.float32)
        + bo_ref[...].reshape(1, 1), 0.0)


def _out_stage(agg0, agg1, h2, Wl_bd, Wr_bd, g_c, be_c, Wo_bd, b_o):
    return pl.pallas_call(
        _out_body,
        out_shape=(jax.ShapeDtypeStruct((NP, 128), jnp.float32),
                   jax.ShapeDtypeStruct((NP, P), jnp.float32)),
    )(agg0, agg1, h2, Wl_bd, Wr_bd, g_c, be_c, Wo_bd, b_o)


def kernel(x, seq, pause, edge_index, W_x, b_x, g_x, be_x, W_p, b_p, g_p, be_p,
           W_e, b_e, g_e, be_e, W_f, b_f, g_f, be_f, W_l, b_l, W_r, g_c, be_c,
           W_o, b_o):
    eye4 = jnp.eye(P, dtype=jnp.float32)
    Wf_t_bd = jnp.kron(eye4, W_f[:H])
    Wf_b_bd = jnp.kron(eye4, W_f[H:])
    Wl_bd = jnp.kron(eye4, W_l)
    Wr_bd = jnp.kron(eye4, W_r)
    Wo_bd = jnp.kron(eye4, W_o)

    A = _encoder_gemm(seq, W_e.astype(jnp.bfloat16)).reshape(NP, 128)

    xp = jnp.broadcast_to(x.reshape(NP, P, 1), (NP, P, H)).reshape(NP, 128)
    pp = jnp.broadcast_to(pause.reshape(NP, P, 1), (NP, P, H)).reshape(NP, 128)

    h2 = _mid_stage(A, xp, pp, W_x, g_x, be_x, W_p, g_p, be_p, g_e, be_e,
                    Wf_t_bd, Wf_b_bd, g_f, be_f)

    zeros = jnp.zeros((N, H), jnp.float32)
    agg0, agg1 = _sc_scatter(h2.reshape(N, H), edge_index, zeros)

    z, out = _out_stage(agg0.reshape(NP, 128), agg1.reshape(NP, 128), h2,
                        Wl_bd, Wr_bd, g_c, be_c, Wo_bd, b_o)
    return (out.reshape(N, 1), z.reshape(N, H))
